# baseline (speedup 1.0000x reference)
import jax
import jax.numpy as jnp
from jax import lax
from jax.experimental import pallas as pl

_NUM_PIECES = 32


def _extract_block(ids_ref, board_ref, out_ref):
    ids = ids_ref[...]
    bb, hw_n = ids.shape
    t = lax.broadcasted_iota(jnp.int32, (bb, _NUM_PIECES, hw_n), 1) + 1
    hw = lax.broadcasted_iota(jnp.int32, (bb, _NUM_PIECES, hw_n), 2)
    mask = ids[:, None, :] == t
    masked_pos = jnp.where(mask, hw, hw_n)
    first = jnp.min(masked_pos, axis=2, keepdims=True)
    onehot = (hw == first).astype(jnp.float32)
    board = board_ref[...]
    out_ref[...] = lax.dot_general(
        onehot, board,
        dimension_numbers=(((2,), (2,)), ((0,), (0,))),
        preferred_element_type=jnp.float32,
    )


def kernel(full_board_vector, piece_ids):
    B, C, H, W = full_board_vector.shape
    HW = H * W
    flat_ids = piece_ids.reshape(B, HW)
    flat_board = full_board_vector.reshape(B, C, HW)

    BB = 128
    grid = (B // BB,)
    return pl.pallas_call(
        _extract_block,
        grid=grid,
        in_specs=[
            pl.BlockSpec((BB, HW), lambda i: (i, 0)),
            pl.BlockSpec((BB, C, HW), lambda i: (i, 0, 0)),
        ],
        out_specs=pl.BlockSpec((BB, _NUM_PIECES, C), lambda i: (i, 0, 0)),
        out_shape=jax.ShapeDtypeStruct((B, _NUM_PIECES, C), jnp.float32),
    )(flat_ids, flat_board)
